# Initial kernel scaffold; baseline (speedup 1.0000x reference)
import jax
import jax.numpy as jnp
from jax.experimental import pallas as pl


def kernel(x, edge_index, params):
    raise NotImplementedError("write your pallas kernel here")


# speedup vs baseline: 2.2086x; 2.2086x over previous
import functools

import numpy as np
import jax
import jax.numpy as jnp
from jax import lax
from jax.experimental import pallas as pl
from jax.experimental.pallas import tpu as pltpu
from jax.experimental.pallas import tpu_sc as plsc

N_NODES = 50000
N_EDGES = 800000
H = 32
TW = 48
VW = 16

NC, NS = 2, 16
NW = NC * NS
EPW = N_EDGES // NW
CH = 128
NFULL = EPW // CH
REM = EPW - NFULL * CH
NPT = N_NODES // NS
ZR = 625

EB = 4000
NB = 2000

_pallas_call = pl.pallas_call
_I0 = np.int32(0)


def _mesh():
    return plsc.VectorSubcoreMesh(
        core_axis_name="c", subcore_axis_name="s",
        num_cores=NC, num_subcores=NS)


def _sc_gather_pair(table, idx_i, idx_j):
    @functools.partial(
        pl.kernel,
        out_type=(jax.ShapeDtypeStruct((N_EDGES, TW), jnp.float32),
                  jax.ShapeDtypeStruct((N_EDGES, TW), jnp.float32)),
        mesh=_mesh(),
        compiler_params=pltpu.CompilerParams(use_tc_tiling_on_sc=False),
        scratch_types=[
            pltpu.VMEM((CH,), jnp.int32),
            pltpu.VMEM((CH, TW), jnp.float32),
            pltpu.VMEM((REM,), jnp.int32),
            pltpu.VMEM((REM, TW), jnp.float32),
            pltpu.SemaphoreType.DMA,
        ],
    )
    def k(table_h, ii_h, ij_h, oi_h, oj_h, idx_v, rows_v, idxr_v, rowsr_v, sem):
        wid = lax.axis_index("c") * NS + lax.axis_index("s")
        base = wid * EPW

        def chunk(off, in_h, out_h, iv, rv, n):
            pltpu.sync_copy(in_h.at[pl.ds(off, n)], iv)
            pltpu.async_copy(table_h.at[iv], rv, sem).wait()
            pltpu.sync_copy(rv, out_h.at[pl.ds(off, n), :])

        def body(g, carry):
            off = base + g * CH
            chunk(off, ii_h, oi_h, idx_v, rows_v, CH)
            chunk(off, ij_h, oj_h, idx_v, rows_v, CH)
            return carry

        lax.fori_loop(jnp.int32(0), jnp.int32(NFULL), body, jnp.int32(0))
        off = base + NFULL * CH
        chunk(off, ii_h, oi_h, idxr_v, rowsr_v, REM)
        chunk(off, ij_h, oj_h, idxr_v, rowsr_v, REM)

    return k(table, idx_i, idx_j)


def _sc_scatter_add(idx_i, vals, zeros_rows):
    D = vals.shape[1]

    @functools.partial(
        pl.kernel,
        out_type=jax.ShapeDtypeStruct((NC, N_NODES, D), jnp.float32),
        mesh=_mesh(),
        compiler_params=pltpu.CompilerParams(use_tc_tiling_on_sc=False),
        scratch_types=[
            pltpu.VMEM((CH,), jnp.int32),
            pltpu.VMEM((CH, D), jnp.float32),
            pltpu.VMEM((REM,), jnp.int32),
            pltpu.VMEM((REM, D), jnp.float32),
            pltpu.VMEM_SHARED((N_NODES, D), jnp.float32),
        ],
    )
    def k(idx_h, vals_h, zer_h, out_h, idx_v, vals_v, idxr_v, valsr_v, acc):
        cid = lax.axis_index("c")
        sid = lax.axis_index("s")
        wid = cid * NS + sid

        def zbody(t, carry):
            pltpu.sync_copy(zer_h, acc.at[pl.ds(sid * NPT + t * ZR, ZR), :])
            return carry

        lax.fori_loop(jnp.int32(0), jnp.int32(NPT // ZR), zbody, jnp.int32(0))
        plsc.subcore_barrier()

        base = wid * EPW

        def chunk(off, iv, vv, n):
            pltpu.sync_copy(idx_h.at[pl.ds(off, n)], iv)
            pltpu.sync_copy(vals_h.at[pl.ds(off, n), :], vv)
            pltpu.sync_copy(vv, acc.at[iv], add=True)

        def body(g, carry):
            chunk(base + g * CH, idx_v, vals_v, CH)
            return carry

        lax.fori_loop(jnp.int32(0), jnp.int32(NFULL), body, jnp.int32(0))
        chunk(base + NFULL * CH, idxr_v, valsr_v, REM)
        plsc.subcore_barrier()
        pltpu.sync_copy(acc.at[pl.ds(sid * NPT, NPT), :],
                        out_h.at[cid, pl.ds(sid * NPT, NPT), :])

    return k(idx_i, vals, zeros_rows)


def _ln(h, g, bb):
    mu = jnp.mean(h, axis=-1, keepdims=True)
    var = jnp.mean((h - mu) * (h - mu), axis=-1, keepdims=True)
    return (h - mu) * lax.rsqrt(var + 1e-5) * g + bb


def _psi(t):
    return jnp.sign(t) * jnp.log(jnp.abs(t) + 1.0)


def _wspec(a):
    return pl.BlockSpec(a.shape, lambda i: (_I0,) * a.ndim)


def _r2(a):
    return a.reshape(1, -1)


def _node_encode(x, w):
    def body(x_r, w0, b0, g0, bb0, W1, b1, t_r):
        xb = x_r[...]
        s0 = xb[:, 2:3]
        h = s0 * w0[...] + b0[...]
        h = jnp.maximum(_ln(h, g0[...], bb0[...]), 0.0)
        s = h @ W1[...] + b1[...]
        v = xb[:, 0:2]
        pad = jnp.zeros((NB, TW - H - 2), jnp.float32)
        t_r[...] = jnp.concatenate([s, v, pad], axis=1)

    return _pallas_call(
        body,
        grid=(N_NODES // NB,),
        in_specs=[pl.BlockSpec((NB, 3), lambda i: (i, _I0))] + [_wspec(a) for a in w],
        out_specs=pl.BlockSpec((NB, TW), lambda i: (i, _I0)),
        out_shape=jax.ShapeDtypeStruct((N_NODES, TW), jnp.float32),
    )(x, *w)


def _edge_step(gi, gj, e_in, weights, first):
    nw = len(weights)

    def body(*refs):
        gi_r, gj_r = refs[0], refs[1]
        p = 2
        if not first:
            e_r = refs[p]
            p += 1
        wr = [refs[p + t] for t in range(nw)]
        m_ref, upd_ref, eo_ref = refs[p + nw:]
        it = iter(wr)

        def nx():
            return next(it)[...]

        gib = gi_r[...]
        gjb = gj_r[...]
        si = gib[:, 0:H]
        sj = gjb[:, 0:H]
        vi = gib[:, H:H + VW]
        vj = gjb[:, H:H + VW]
        vd = vi - vj
        norms = _psi(jnp.sum(vd * vd, axis=1, keepdims=True))
        dots = _psi(jnp.sum(vi * vj, axis=1, keepdims=True))

        if first:
            h = si @ nx() + sj @ nx() + nx()
            h = jnp.maximum(_ln(h, nx(), nx()), 0.0)
            e = h @ nx() + nx()
        else:
            e = e_r[...]

        h = norms * nx() + dots * nx() + si @ nx() + sj @ nx() + e @ nx() + nx()
        h = jnp.maximum(_ln(h, nx(), nx()), 0.0)
        e_ij = h @ nx() + nx()

        h = e_ij @ nx() + nx()
        h = jnp.maximum(_ln(h, nx(), nx()), 0.0)
        h = h @ nx() + nx()
        m = jax.nn.sigmoid(_ln(h, nx(), nx()))

        h = m @ nx() + nx()
        h = jnp.maximum(_ln(h, nx(), nx()), 0.0)
        h = h @ nx() + nx()
        h = jnp.maximum(_ln(h, nx(), nx()), 0.0)
        px = h @ nx() + nx()

        upd = jnp.clip(vd * px, -100.0, 100.0)
        m_ref[...] = m
        upd_ref[...] = upd
        eo_ref[...] = e_ij

    ins = [gi, gj] + ([] if first else [e_in]) + list(weights)
    in_specs = [pl.BlockSpec((EB, TW), lambda i: (i, _I0)),
                pl.BlockSpec((EB, TW), lambda i: (i, _I0))]
    if not first:
        in_specs.append(pl.BlockSpec((EB, H), lambda i: (i, _I0)))
    in_specs += [_wspec(a) for a in weights]

    return _pallas_call(
        body,
        grid=(N_EDGES // EB,),
        in_specs=in_specs,
        out_specs=[pl.BlockSpec((EB, H), lambda i: (i, _I0)),
                   pl.BlockSpec((EB, VW), lambda i: (i, _I0)),
                   pl.BlockSpec((EB, H), lambda i: (i, _I0))],
        out_shape=[jax.ShapeDtypeStruct((N_EDGES, H), jnp.float32),
                   jax.ShapeDtypeStruct((N_EDGES, VW), jnp.float32),
                   jax.ShapeDtypeStruct((N_EDGES, H), jnp.float32)],
    )(*ins)


def _node_update(t, sp0, sp1, vp0, vp1, w):
    def body(t_r, sp0_r, sp1_r, vp0_r, vp1_r, Ws, Wa, b, g, bb, W2, b2, to_r):
        tb = t_r[...]
        s = tb[:, 0:H]
        v16 = tb[:, H:TW]
        sagg = sp0_r[...] + sp1_r[...]
        vagg = vp0_r[...] + vp1_r[...]
        h = s @ Ws[...] + sagg @ Wa[...] + b[...]
        h = jnp.maximum(_ln(h, g[...], bb[...]), 0.0)
        ds = h @ W2[...] + b2[...]
        to_r[...] = jnp.concatenate([s + ds, v16 + vagg], axis=1)

    return _pallas_call(
        body,
        grid=(N_NODES // NB,),
        in_specs=[pl.BlockSpec((NB, TW), lambda i: (i, _I0)),
                  pl.BlockSpec((NB, H), lambda i: (i, _I0)),
                  pl.BlockSpec((NB, H), lambda i: (i, _I0)),
                  pl.BlockSpec((NB, VW), lambda i: (i, _I0)),
                  pl.BlockSpec((NB, VW), lambda i: (i, _I0))]
                 + [_wspec(a) for a in w],
        out_specs=pl.BlockSpec((NB, TW), lambda i: (i, _I0)),
        out_shape=jax.ShapeDtypeStruct((N_NODES, TW), jnp.float32),
    )(t, sp0, sp1, vp0, vp1, *w)


def _edge_final(gi, gj, e, w):
    def body(gi_r, gj_r, e_r, Wsj, Wsi, We, b1, g1, bb1, W2, b2, g2, bb2,
             w3, b3, o_r):
        si = gi_r[...][:, 0:H]
        sj = gj_r[...][:, 0:H]
        h = sj @ Wsj[...] + si @ Wsi[...] + e_r[...] @ We[...] + b1[...]
        h = jnp.maximum(_ln(h, g1[...], bb1[...]), 0.0)
        h = h @ W2[...] + b2[...]
        h = jnp.maximum(_ln(h, g2[...], bb2[...]), 0.0)
        o_r[...] = jax.nn.sigmoid(h @ w3[...] + b3[...])

    return _pallas_call(
        body,
        grid=(N_EDGES // EB,),
        in_specs=[pl.BlockSpec((EB, TW), lambda i: (i, _I0)),
                  pl.BlockSpec((EB, TW), lambda i: (i, _I0)),
                  pl.BlockSpec((EB, H), lambda i: (i, _I0))]
                 + [_wspec(a) for a in w],
        out_specs=pl.BlockSpec((EB, 1), lambda i: (i, _I0)),
        out_shape=jax.ShapeDtypeStruct((N_EDGES, 1), jnp.float32),
    )(gi, gj, e, *w)


def _flatten_params(params):
    ne = params["node_encoder"]
    w_ne = (_r2(ne[0]["W"]), _r2(ne[0]["b"]), _r2(ne[0]["g"]), _r2(ne[0]["bb"]),
            ne[1]["W"], _r2(ne[1]["b"]))

    ee = params["edge_encoder"]
    W = ee[0]["W"]
    w_enc = (W[0:H], W[H:2 * H], _r2(ee[0]["b"]), _r2(ee[0]["g"]),
             _r2(ee[0]["bb"]), ee[1]["W"], _r2(ee[1]["b"]))

    pe = params["phi_e"]
    W = pe[0]["W"]
    w_pe = (_r2(W[0]), _r2(W[1]), W[2:2 + H], W[2 + H:2 + 2 * H],
            W[2 + 2 * H:2 + 3 * H], _r2(pe[0]["b"]), _r2(pe[0]["g"]),
            _r2(pe[0]["bb"]), pe[1]["W"], _r2(pe[1]["b"]))

    pm = params["phi_m"]
    w_pm = (pm[0]["W"], _r2(pm[0]["b"]), _r2(pm[0]["g"]), _r2(pm[0]["bb"]),
            pm[1]["W"], _r2(pm[1]["b"]), _r2(pm[1]["g"]), _r2(pm[1]["bb"]))

    px = params["phi_x"]
    w_px = (px[0]["W"], _r2(px[0]["b"]), _r2(px[0]["g"]), _r2(px[0]["bb"]),
            px[1]["W"], _r2(px[1]["b"]), _r2(px[1]["g"]), _r2(px[1]["bb"]),
            px[2]["W"], _r2(px[2]["b"]))

    ps = params["phi_s"]
    W = ps[0]["W"]
    w_ps = (W[0:H], W[H:2 * H], _r2(ps[0]["b"]), _r2(ps[0]["g"]),
            _r2(ps[0]["bb"]), ps[1]["W"], _r2(ps[1]["b"]))

    em = params["edge_mlp"]
    W = em[0]["W"]
    w_em = (W[0:H], W[H:2 * H], W[2 * H:3 * H], _r2(em[0]["b"]),
            _r2(em[0]["g"]), _r2(em[0]["bb"]), em[1]["W"], _r2(em[1]["b"]),
            _r2(em[1]["g"]), _r2(em[1]["bb"]), em[2]["W"], _r2(em[2]["b"]))

    return w_ne, w_enc, w_pe, w_pm, w_px, w_ps, w_em


def kernel(x, edge_index, params):
    idx2 = edge_index.astype(jnp.int32)
    idx_i = idx2[0]
    idx_j = idx2[1]
    w_ne, w_enc, w_pe, w_pm, w_px, w_ps, w_em = _flatten_params(params)
    z32 = jnp.zeros((ZR, H), jnp.float32)
    z16 = jnp.zeros((ZR, VW), jnp.float32)

    t = _node_encode(x.astype(jnp.float32), w_ne)
    e = None
    for step in range(2):
        gi, gj = _sc_gather_pair(t, idx_i, idx_j)
        if step == 0:
            m, upd, e = _edge_step(gi, gj, None, w_enc + w_pe + w_pm + w_px,
                                   first=True)
        else:
            m, upd, e = _edge_step(gi, gj, e, w_pe + w_pm + w_px, first=False)
        sp = _sc_scatter_add(idx_i, m, z32)
        vp = _sc_scatter_add(idx_i, upd, z16)
        t = _node_update(t, sp[0], sp[1], vp[0], vp[1], w_ps)

    gi, gj = _sc_gather_pair(t, idx_i, idx_j)
    return _edge_final(gi, gj, e, w_em)
